# baseline (speedup 1.0000x reference)
import jax
import jax.numpy as jnp
from jax import lax
from jax.experimental import pallas as pl
from jax.experimental.pallas import tpu as pltpu
from jax.experimental.pallas import tpu_sc as plsc

B, T, D, V = 4096, 200, 64, 1000000
DP = 128
N = B * T
NC, NS = 2, 16
NW = NC * NS
PER_W = N // NW
CHUNK = 128
NCH = PER_W // CHUNK
K = 2
G = NCH // K


def _emb_body(idx_hbm, table_hbm, out_hbm, idx_v, rows_a, rows_b,
              gsem_a, gsem_b, osem_a, osem_b):
    wid = lax.axis_index("s") * NC + lax.axis_index("c")
    base = wid * PER_W
    pltpu.sync_copy(idx_hbm.at[pl.ds(wid * NCH, NCH)], idx_v)

    def fire_gathers(g, rows_v, gsem):
        for k in range(K):
            c = g * K + k
            pltpu.async_copy(table_hbm.at[idx_v.at[c]],
                             rows_v.at[pl.ds(k * CHUNK, CHUNK)], gsem)

    def wait_gathers(g, rows_v, gsem):
        for k in range(K):
            c = g * K + k
            pltpu.make_async_copy(
                table_hbm.at[idx_v.at[c]],
                rows_v.at[pl.ds(k * CHUNK, CHUNK)], gsem).wait()

    def fire_outs(g, rows_v, osem):
        for k in range(K):
            c = g * K + k
            pltpu.async_copy(rows_v.at[pl.ds(k * CHUNK, CHUNK)],
                             out_hbm.at[pl.ds(base + c * CHUNK, CHUNK)], osem)

    def drain_outs(rows_v, osem):
        for k in range(K):
            pltpu.make_async_copy(rows_v.at[pl.ds(k * CHUNK, CHUNK)],
                                  out_hbm.at[pl.ds(k * CHUNK, CHUNK)],
                                  osem).wait()

    fire_gathers(0, rows_a, gsem_a)

    def step_pair(gp, _):
        g_even = 2 * gp
        g_odd = 2 * gp + 1
        @pl.when(gp >= 1)
        def _():
            drain_outs(rows_a, osem_a)
            fire_gathers(g_even, rows_a, gsem_a)
            wait_gathers(g_even - 1, rows_b, gsem_b)
            fire_outs(g_even - 1, rows_b, osem_b)
        @pl.when(gp >= 1)
        def _():
            drain_outs(rows_b, osem_b)
        fire_gathers(g_odd, rows_b, gsem_b)
        wait_gathers(g_even, rows_a, gsem_a)
        fire_outs(g_even, rows_a, osem_a)
        return 0

    lax.fori_loop(0, G // 2, step_pair, 0)

    wait_gathers(G - 1, rows_b, gsem_b)
    fire_outs(G - 1, rows_b, osem_b)
    drain_outs(rows_a, osem_a)
    drain_outs(rows_b, osem_b)


def _embedding_lookup(idx_flat, table_padded):
    k = pl.kernel(
        _emb_body,
        out_type=jax.ShapeDtypeStruct((N, DP), jnp.float32),
        mesh=plsc.VectorSubcoreMesh(core_axis_name="c", subcore_axis_name="s"),
        scratch_types=[
            pltpu.VMEM((NCH, CHUNK), jnp.int32),
            pltpu.VMEM((K * CHUNK, DP), jnp.float32),
            pltpu.VMEM((K * CHUNK, DP), jnp.float32),
            pltpu.SemaphoreType.DMA,
            pltpu.SemaphoreType.DMA,
            pltpu.SemaphoreType.DMA,
            pltpu.SemaphoreType.DMA,
        ],
    )
    return k(idx_flat, table_padded)


def kernel(token_ids, subword_emb_weight):
    idx_flat = token_ids.reshape(NW * NCH, CHUNK)
    table_padded = jnp.tile(subword_emb_weight, (1, 2))
    out = _embedding_lookup(idx_flat, table_padded)
    return out[:, :D].reshape(B, T, D)


# speedup vs baseline: 1.1529x; 1.1529x over previous
import jax
import jax.numpy as jnp
from jax import lax
from jax.experimental import pallas as pl
from jax.experimental.pallas import tpu as pltpu
from jax.experimental.pallas import tpu_sc as plsc

B, T, D, V = 4096, 200, 64, 1000000
DP = 128
N = B * T
NC, NS = 2, 16
NW = NC * NS
PER_W = N // NW
CHUNK = 128
NCH = PER_W // CHUNK
K = 2
G = NCH // K


def _emb_body(idx_hbm, table_hbm, out_hbm, idx_v, rows_a, rows_b,
              gsem_a, gsem_b, osem_a, osem_b):
    wid = lax.axis_index("s") * NC + lax.axis_index("c")
    base = wid * PER_W
    pltpu.sync_copy(idx_hbm.at[pl.ds(wid * NCH, NCH)], idx_v)

    def fire_gathers(g, rows_v, gsem):
        for k in range(K):
            c = g * K + k
            pltpu.async_copy(table_hbm.at[idx_v.at[c]],
                             rows_v.at[pl.ds(k * CHUNK, CHUNK)], gsem)

    def wait_gathers(g, rows_v, gsem):
        for k in range(K):
            c = g * K + k
            pltpu.make_async_copy(
                table_hbm.at[idx_v.at[c]],
                rows_v.at[pl.ds(k * CHUNK, CHUNK)], gsem).wait()

    def fire_outs(g, rows_v, osem):
        for k in range(K):
            c = g * K + k
            pltpu.async_copy(rows_v.at[pl.ds(k * CHUNK, CHUNK)],
                             out_hbm.at[pl.ds(base + c * CHUNK, CHUNK)], osem)

    def drain_outs(rows_v, osem):
        for k in range(K):
            pltpu.make_async_copy(rows_v.at[pl.ds(k * CHUNK, CHUNK)],
                                  out_hbm.at[pl.ds(k * CHUNK, CHUNK)],
                                  osem).wait()

    fire_gathers(0, rows_a, gsem_a)

    def step_pair(gp, _):
        g_even = 2 * gp
        g_odd = 2 * gp + 1
        @pl.when(gp >= 1)
        def _():
            drain_outs(rows_a, osem_a)
            fire_gathers(g_even, rows_a, gsem_a)
            wait_gathers(g_even - 1, rows_b, gsem_b)
            fire_outs(g_even - 1, rows_b, osem_b)
        @pl.when(gp >= 1)
        def _():
            drain_outs(rows_b, osem_b)
        fire_gathers(g_odd, rows_b, gsem_b)
        wait_gathers(g_even, rows_a, gsem_a)
        fire_outs(g_even, rows_a, osem_a)
        return 0

    lax.fori_loop(0, G // 2, step_pair, 0)

    wait_gathers(G - 1, rows_b, gsem_b)
    fire_outs(G - 1, rows_b, osem_b)
    drain_outs(rows_a, osem_a)
    drain_outs(rows_b, osem_b)


def _embedding_lookup(idx_flat, table_padded):
    k = pl.kernel(
        _emb_body,
        out_type=jax.ShapeDtypeStruct((N, DP), jnp.float32),
        mesh=plsc.VectorSubcoreMesh(core_axis_name="c", subcore_axis_name="s"),
        scratch_types=[
            pltpu.VMEM((NCH, CHUNK), jnp.int32),
            pltpu.VMEM((K * CHUNK, DP), jnp.float32),
            pltpu.VMEM((K * CHUNK, DP), jnp.float32),
            pltpu.SemaphoreType.DMA,
            pltpu.SemaphoreType.DMA,
            pltpu.SemaphoreType.DMA,
            pltpu.SemaphoreType.DMA,
        ],
    )
    return k(idx_flat, table_padded)


def kernel(token_ids, subword_emb_weight):
    idx_flat = token_ids.reshape(NW * NCH, CHUNK)
    table_padded = jnp.pad(subword_emb_weight.T, ((0, DP - D), (0, 0))).T
    out = _embedding_lookup(idx_flat, table_padded)
    return out[:, :D].reshape(B, T, D)
